# baseline (speedup 1.0000x reference)
import dataclasses
import functools

import jax
import jax.numpy as jnp
import numpy as np
from jax import lax
from jax.experimental import pallas as pl
from jax.experimental.pallas import tpu as pltpu
from jax.experimental.pallas import tpu_sc as plsc

_NC = 2
_NS = 16
_LN = 16
_W = 40
_SW = 50

_PERM = np.arange(128).reshape(4, 2, 16).transpose(0, 2, 1).reshape(-1)


def _mlp_pallas(x_list, layers, relu_flags, block_rows, out_dtype=jnp.float32,
                mm_dtype=None):
    rows = x_list[0].shape[0]
    for x in x_list:
        if x.ndim == 2:
            rows = x.shape[0]
    din = x_list[0].shape[-1]
    n_lay = len(layers)
    dout = layers[-1][0].shape[1]

    n_refs = sum(2 if x.ndim == 3 else 1 for x in x_list)

    def body(*refs):
        x_refs = refs[:n_refs]
        w_refs = refs[n_refs:n_refs + 2 * n_lay]
        acc = None
        for r in x_refs:
            v = r[0] if len(r.shape) == 3 else r[...]
            acc = v if acc is None else acc + v
        for li in range(n_lay):
            w = w_refs[2 * li][...]
            b = w_refs[2 * li + 1][...]
            a = acc
            if mm_dtype is not None:
                a = a.astype(mm_dtype)
                w = w.astype(mm_dtype)
            acc = jnp.dot(a, w, preferred_element_type=jnp.float32) + b
            if relu_flags[li]:
                acc = jnp.maximum(acc, 0.0)
        refs[-1][...] = acc.astype(out_dtype)

    in_specs = []
    flat = []
    for x in x_list:
        if x.ndim == 3:
            for p in range(2):
                in_specs.append(pl.BlockSpec(
                    (1, block_rows, din), lambda i, p=p: (p, i, 0)))
                flat.append(x)
        else:
            in_specs.append(pl.BlockSpec((block_rows, din), lambda i: (i, 0)))
            flat.append(x)
    for w, b in layers:
        in_specs.append(pl.BlockSpec(w.shape, lambda i: (0, 0)))
        in_specs.append(pl.BlockSpec((1, b.shape[0]), lambda i: (0, 0)))
        flat += [w, b.reshape(1, -1)]

    return pl.pallas_call(
        body,
        grid=(rows // block_rows,),
        in_specs=in_specs,
        out_specs=pl.BlockSpec((block_rows, dout), lambda i: (i, 0)),
        out_shape=jax.ShapeDtypeStruct((rows, dout), out_dtype),
    )(*flat)


def _mlp_layers(p):
    return [(p["W1"], p["b1"]), (p["W2"], p["b2"]), (p["W3"], p["b3"])]


def _perm_last(p):
    return (p["W3"][:, _PERM], p["b3"][_PERM])


def _sc_message(hp, ep, src5, dst5):
    n, d = hp.shape
    dw = ep.shape[1]
    nsw = src5.shape[2]
    epw = nsw * _SW * _W
    npad = -(-n // (_NS * 8)) * (_NS * 8)
    rps = npad // _NS
    nfull = rps // _W
    tail = rps - nfull * _W
    mesh = plsc.VectorSubcoreMesh(core_axis_name="c", subcore_axis_name="s")
    cp = pltpu.CompilerParams()
    if "needs_layout_passes" in pltpu.CompilerParams.__dataclass_fields__:
        cp = dataclasses.replace(cp, needs_layout_passes=False)

    @functools.partial(
        pl.kernel,
        out_type=jax.ShapeDtypeStruct((_NC, npad, d), jnp.float32),
        mesh=mesh,
        compiler_params=cp,
        scratch_types=[
            pltpu.VMEM((_SW, _W), jnp.int32),
            pltpu.VMEM((_SW, _W), jnp.int32),
            pltpu.VMEM((_W, d), jnp.float32),
            pltpu.VMEM((_W, dw), jnp.int32),
            pltpu.VMEM((_W, d), jnp.float32),
            pltpu.VMEM((_W, dw), jnp.int32),
            pltpu.VMEM_SHARED((npad, d), jnp.float32),
            pltpu.SemaphoreType.DMA,
            pltpu.SemaphoreType.DMA,
            pltpu.SemaphoreType.DMA,
            pltpu.SemaphoreType.DMA,
        ],
    )
    def k(h_hbm, e_hbm, src_hbm, dst_hbm, out_hbm,
          src_v, dst_v, h_a, e_a, h_b, e_b, agg,
          sem_ha, sem_ea, sem_hb, sem_eb):
        c = lax.axis_index("c")
        s = lax.axis_index("s")
        wbase = (c * _NS + s) * epw

        @pl.loop(0, _W)
        def _(i):
            for j in range(d // _LN):
                h_a[i, pl.ds(j * _LN, _LN)] = jnp.zeros((_LN,), jnp.float32)

        @pl.loop(0, nfull)
        def _(q):
            pltpu.async_copy(
                h_a, agg.at[pl.ds(s * rps + q * _W, _W), :], sem_ha)
        if tail:
            pltpu.async_copy(
                h_a.at[pl.ds(0, tail), :],
                agg.at[pl.ds(s * rps + nfull * _W, tail), :], sem_ea)

        @pl.loop(0, nfull)
        def _(q):
            pltpu.make_async_copy(
                h_a, agg.at[pl.ds(s * rps, _W), :], sem_ha).wait()
        if tail:
            pltpu.make_async_copy(
                h_a.at[pl.ds(0, tail), :],
                agg.at[pl.ds(s * rps, tail), :], sem_ea).wait()

        plsc.subcore_barrier()

        mask = jnp.full((16,), -65536, jnp.int32)

        def fire(sw, g, hbuf, ebuf, sem_h, sem_e):
            pltpu.async_copy(h_hbm.at[src_v.at[g]], hbuf, sem_h)
            pltpu.async_copy(
                e_hbm.at[pl.ds(wbase + (sw * _SW + g) * _W, _W), :],
                ebuf, sem_e)

        def drain(hbuf, ebuf, sem_h, sem_e):
            pltpu.make_async_copy(h_hbm.at[pl.ds(0, _W), :], hbuf, sem_h).wait()
            pltpu.make_async_copy(e_hbm.at[pl.ds(0, _W), :], ebuf, sem_e).wait()

        def compute_scatter(g, hbuf, ebuf):
            @plsc.parallel_loop(0, _W)
            def _(i):
                for j in range(d // 32):
                    w = ebuf[i, pl.ds(j * _LN, _LN)]
                    lo = plsc.bitcast(jnp.left_shift(w, 16), jnp.float32)
                    hi = plsc.bitcast(jnp.bitwise_and(w, mask), jnp.float32)
                    sl = pl.ds(j * 32, _LN)
                    sh = pl.ds(j * 32 + _LN, _LN)
                    hbuf[i, sl] = jnp.maximum(hbuf[i, sl] + lo, 0.0)
                    hbuf[i, sh] = jnp.maximum(hbuf[i, sh] + hi, 0.0)
            pltpu.sync_copy(hbuf, agg.at[dst_v.at[g]], add=True)

        @pl.loop(0, nsw)
        def _(sw):
            pltpu.sync_copy(src_hbm.at[c, s, sw], src_v)
            pltpu.sync_copy(dst_hbm.at[c, s, sw], dst_v)
            fire(sw, 0, h_a, e_a, sem_ha, sem_ea)
            fire(sw, 1, h_b, e_b, sem_hb, sem_eb)

            @pl.loop(0, _SW, step=2)
            def _(g):
                drain(h_a, e_a, sem_ha, sem_ea)
                compute_scatter(g, h_a, e_a)

                @pl.when(g + 2 < _SW)
                def _():
                    fire(sw, g + 2, h_a, e_a, sem_ha, sem_ea)

                drain(h_b, e_b, sem_hb, sem_eb)
                compute_scatter(g + 1, h_b, e_b)

                @pl.when(g + 3 < _SW)
                def _():
                    fire(sw, g + 3, h_b, e_b, sem_hb, sem_eb)

        plsc.subcore_barrier()

        @pl.loop(0, nfull)
        def _(q):
            base = s * rps + q * _W
            pltpu.async_copy(agg.at[pl.ds(base, _W), :],
                             out_hbm.at[c, pl.ds(base, _W), :], sem_ha)
        if tail:
            base = s * rps + nfull * _W
            pltpu.async_copy(agg.at[pl.ds(base, tail), :],
                             out_hbm.at[c, pl.ds(base, tail), :], sem_ea)

        @pl.loop(0, nfull)
        def _(q):
            pltpu.make_async_copy(agg.at[pl.ds(s * rps, _W), :],
                                  out_hbm.at[c, pl.ds(s * rps, _W), :],
                                  sem_ha).wait()
        if tail:
            pltpu.make_async_copy(agg.at[pl.ds(s * rps, tail), :],
                                  out_hbm.at[c, pl.ds(s * rps, tail), :],
                                  sem_ea).wait()

    return k(hp, ep, src5, dst5)


def kernel(x, edge_index, edge_attr, params):
    e_cnt = edge_attr.shape[0]
    nsw = e_cnt // (_NC * _NS * _SW * _W)
    src5 = edge_index[0].astype(jnp.int32).reshape(_NC, _NS, nsw, _SW, _W)
    dst5 = edge_index[1].astype(jnp.int32).reshape(_NC, _NS, nsw, _SW, _W)

    h = _mlp_pallas([x], _mlp_layers(params["node_to_node"]),
                    (True, True, False), block_rows=2000)
    ep = _mlp_pallas([edge_attr], _mlp_layers(params["edge_to_node"])[:2]
                     + [_perm_last(params["edge_to_node"])],
                     (True, True, False), block_rows=2000,
                     out_dtype=jnp.bfloat16, mm_dtype=jnp.bfloat16)
    ep = lax.bitcast_convert_type(
        ep.reshape(e_cnt, ep.shape[1] // 2, 2), jnp.int32)

    parts = _sc_message(h, ep, src5, dst5)
    h = _mlp_pallas([h, parts], _mlp_layers(params["gine"][0]),
                    (True, True, False), block_rows=2000)

    parts = _sc_message(h, ep, src5, dst5)
    return _mlp_pallas(
        [h, parts],
        _mlp_layers(params["gine"][1]) + _mlp_layers(params["final_mlp"]),
        (True, True, False, True, True, False), block_rows=2000)


# speedup vs baseline: 1.9966x; 1.9966x over previous
import dataclasses
import functools

import jax
import jax.numpy as jnp
import numpy as np
from jax import lax
from jax.experimental import pallas as pl
from jax.experimental.pallas import tpu as pltpu
from jax.experimental.pallas import tpu_sc as plsc

_NC = 2
_NS = 16
_LN = 16
_W = 40
_SW = 50

_IDX = np.arange(128).reshape(4, 2, 16)
_PLO = _IDX[:, 0, :].reshape(-1)
_PHI = _IDX[:, 1, :].reshape(-1)


def _round_bf16_bits(v32):
    return v32 + 32767 + jnp.bitwise_and(jnp.right_shift(v32, 16), 1)


def _mlp_pallas(x_list, layers, relu_flags, block_rows, out_dtype=jnp.float32,
                mm_dtype=None, pack_out=None):
    rows = x_list[0].shape[0]
    for x in x_list:
        if x.ndim == 2:
            rows = x.shape[0]
    din = x_list[0].shape[-1]
    n_lay = len(layers)
    if pack_out is not None:
        dout = pack_out[0].shape[1]
        out_dtype = jnp.int32
    else:
        dout = layers[-1][0].shape[1]

    n_refs = sum(2 if x.ndim == 3 else 1 for x in x_list)
    wmats = [(w, b) for w, b in layers]
    if pack_out is not None:
        wmats += [(pack_out[0], pack_out[1]), (pack_out[2], pack_out[3])]

    def body(*refs):
        x_refs = refs[:n_refs]
        w_refs = refs[n_refs:n_refs + 2 * len(wmats)]
        acc = None
        for r in x_refs:
            v = r[0] if len(r.shape) == 3 else r[...]
            acc = v if acc is None else acc + v

        def mat(a, li):
            w = w_refs[2 * li][...]
            b = w_refs[2 * li + 1][...]
            if mm_dtype is not None:
                a = a.astype(mm_dtype)
                w = w.astype(mm_dtype)
            return jnp.dot(a, w, preferred_element_type=jnp.float32) + b

        for li in range(n_lay):
            acc = mat(acc, li)
            if relu_flags[li]:
                acc = jnp.maximum(acc, 0.0)
        if pack_out is not None:
            lo = jax.lax.bitcast_convert_type(mat(acc, n_lay), jnp.int32)
            hi = jax.lax.bitcast_convert_type(mat(acc, n_lay + 1), jnp.int32)
            lo_bits = jnp.bitwise_and(
                jnp.right_shift(_round_bf16_bits(lo), 16), 65535)
            hi_bits = jnp.bitwise_and(_round_bf16_bits(hi), -65536)
            refs[-1][...] = jnp.bitwise_or(lo_bits, hi_bits)
        else:
            refs[-1][...] = acc.astype(out_dtype)

    in_specs = []
    flat = []
    for x in x_list:
        if x.ndim == 3:
            for p in range(2):
                in_specs.append(pl.BlockSpec(
                    (1, block_rows, din), lambda i, p=p: (p, i, 0)))
                flat.append(x)
        else:
            in_specs.append(pl.BlockSpec((block_rows, din), lambda i: (i, 0)))
            flat.append(x)
    for w, b in wmats:
        in_specs.append(pl.BlockSpec(w.shape, lambda i: (0, 0)))
        in_specs.append(pl.BlockSpec((1, b.shape[0]), lambda i: (0, 0)))
        flat += [w, b.reshape(1, -1)]

    return pl.pallas_call(
        body,
        grid=(rows // block_rows,),
        in_specs=in_specs,
        out_specs=pl.BlockSpec((block_rows, dout), lambda i: (i, 0)),
        out_shape=jax.ShapeDtypeStruct((rows, dout), out_dtype),
    )(*flat)


def _mlp_layers(p):
    return [(p["W1"], p["b1"]), (p["W2"], p["b2"]), (p["W3"], p["b3"])]


def _pack_last(p):
    return (p["W3"][:, _PLO], p["b3"][_PLO],
            p["W3"][:, _PHI], p["b3"][_PHI])


def _sc_message(hp, ep, src5, dst5):
    n, d = hp.shape
    dw = ep.shape[1]
    nsw = src5.shape[2]
    epw = nsw * _SW * _W
    npad = -(-n // (_NS * 8)) * (_NS * 8)
    rps = npad // _NS
    nfull = rps // _W
    tail = rps - nfull * _W
    mesh = plsc.VectorSubcoreMesh(core_axis_name="c", subcore_axis_name="s")
    cp = pltpu.CompilerParams()
    if "needs_layout_passes" in pltpu.CompilerParams.__dataclass_fields__:
        cp = dataclasses.replace(cp, needs_layout_passes=False)

    @functools.partial(
        pl.kernel,
        out_type=jax.ShapeDtypeStruct((_NC, npad, d), jnp.float32),
        mesh=mesh,
        compiler_params=cp,
        scratch_types=[
            pltpu.VMEM((_SW, _W), jnp.int32),
            pltpu.VMEM((_SW, _W), jnp.int32),
            pltpu.VMEM((_W, d), jnp.float32),
            pltpu.VMEM((_W, dw), jnp.int32),
            pltpu.VMEM((_W, d), jnp.float32),
            pltpu.VMEM((_W, dw), jnp.int32),
            pltpu.VMEM_SHARED((npad, d), jnp.float32),
            pltpu.SemaphoreType.DMA,
            pltpu.SemaphoreType.DMA,
            pltpu.SemaphoreType.DMA,
            pltpu.SemaphoreType.DMA,
        ],
    )
    def k(h_hbm, e_hbm, src_hbm, dst_hbm, out_hbm,
          src_v, dst_v, h_a, e_a, h_b, e_b, agg,
          sem_ha, sem_ea, sem_hb, sem_eb):
        c = lax.axis_index("c")
        s = lax.axis_index("s")
        wbase = (c * _NS + s) * epw

        @pl.loop(0, _W)
        def _(i):
            for j in range(d // _LN):
                h_a[i, pl.ds(j * _LN, _LN)] = jnp.zeros((_LN,), jnp.float32)

        @pl.loop(0, nfull)
        def _(q):
            pltpu.async_copy(
                h_a, agg.at[pl.ds(s * rps + q * _W, _W), :], sem_ha)
        if tail:
            pltpu.async_copy(
                h_a.at[pl.ds(0, tail), :],
                agg.at[pl.ds(s * rps + nfull * _W, tail), :], sem_ea)

        @pl.loop(0, nfull)
        def _(q):
            pltpu.make_async_copy(
                h_a, agg.at[pl.ds(s * rps, _W), :], sem_ha).wait()
        if tail:
            pltpu.make_async_copy(
                h_a.at[pl.ds(0, tail), :],
                agg.at[pl.ds(s * rps, tail), :], sem_ea).wait()

        plsc.subcore_barrier()

        mask = jnp.full((16,), -65536, jnp.int32)

        def fire(sw, g, hbuf, ebuf, sem_h, sem_e):
            pltpu.async_copy(h_hbm.at[src_v.at[g]], hbuf, sem_h)
            pltpu.async_copy(
                e_hbm.at[pl.ds(wbase + (sw * _SW + g) * _W, _W), :],
                ebuf, sem_e)

        def drain(hbuf, ebuf, sem_h, sem_e):
            pltpu.make_async_copy(h_hbm.at[pl.ds(0, _W), :], hbuf, sem_h).wait()
            pltpu.make_async_copy(e_hbm.at[pl.ds(0, _W), :], ebuf, sem_e).wait()

        def compute_scatter(g, hbuf, ebuf):
            @plsc.parallel_loop(0, _W)
            def _(i):
                for j in range(d // 32):
                    w = ebuf[i, pl.ds(j * _LN, _LN)]
                    lo = plsc.bitcast(jnp.left_shift(w, 16), jnp.float32)
                    hi = plsc.bitcast(jnp.bitwise_and(w, mask), jnp.float32)
                    sl = pl.ds(j * 32, _LN)
                    sh = pl.ds(j * 32 + _LN, _LN)
                    hbuf[i, sl] = jnp.maximum(hbuf[i, sl] + lo, 0.0)
                    hbuf[i, sh] = jnp.maximum(hbuf[i, sh] + hi, 0.0)
            pltpu.sync_copy(hbuf, agg.at[dst_v.at[g]], add=True)

        @pl.loop(0, nsw)
        def _(sw):
            pltpu.sync_copy(src_hbm.at[c, s, sw], src_v)
            pltpu.sync_copy(dst_hbm.at[c, s, sw], dst_v)
            fire(sw, 0, h_a, e_a, sem_ha, sem_ea)
            fire(sw, 1, h_b, e_b, sem_hb, sem_eb)

            @pl.loop(0, _SW, step=2)
            def _(g):
                drain(h_a, e_a, sem_ha, sem_ea)
                compute_scatter(g, h_a, e_a)

                @pl.when(g + 2 < _SW)
                def _():
                    fire(sw, g + 2, h_a, e_a, sem_ha, sem_ea)

                drain(h_b, e_b, sem_hb, sem_eb)
                compute_scatter(g + 1, h_b, e_b)

                @pl.when(g + 3 < _SW)
                def _():
                    fire(sw, g + 3, h_b, e_b, sem_hb, sem_eb)

        plsc.subcore_barrier()

        @pl.loop(0, nfull)
        def _(q):
            base = s * rps + q * _W
            pltpu.async_copy(agg.at[pl.ds(base, _W), :],
                             out_hbm.at[c, pl.ds(base, _W), :], sem_ha)
        if tail:
            base = s * rps + nfull * _W
            pltpu.async_copy(agg.at[pl.ds(base, tail), :],
                             out_hbm.at[c, pl.ds(base, tail), :], sem_ea)

        @pl.loop(0, nfull)
        def _(q):
            pltpu.make_async_copy(agg.at[pl.ds(s * rps, _W), :],
                                  out_hbm.at[c, pl.ds(s * rps, _W), :],
                                  sem_ha).wait()
        if tail:
            pltpu.make_async_copy(agg.at[pl.ds(s * rps, tail), :],
                                  out_hbm.at[c, pl.ds(s * rps, tail), :],
                                  sem_ea).wait()

    return k(hp, ep, src5, dst5)


def kernel(x, edge_index, edge_attr, params):
    e_cnt = edge_attr.shape[0]
    nsw = e_cnt // (_NC * _NS * _SW * _W)
    src5 = edge_index[0].astype(jnp.int32).reshape(_NC, _NS, nsw, _SW, _W)
    dst5 = edge_index[1].astype(jnp.int32).reshape(_NC, _NS, nsw, _SW, _W)

    h = _mlp_pallas([x], _mlp_layers(params["node_to_node"]),
                    (True, True, False), block_rows=2000)
    ep = _mlp_pallas([edge_attr], _mlp_layers(params["edge_to_node"])[:2],
                     (True, True), block_rows=2000,
                     mm_dtype=jnp.bfloat16,
                     pack_out=_pack_last(params["edge_to_node"]))

    parts = _sc_message(h, ep, src5, dst5)
    h = _mlp_pallas([h, parts], _mlp_layers(params["gine"][0]),
                    (True, True, False), block_rows=2000)

    parts = _sc_message(h, ep, src5, dst5)
    return _mlp_pallas(
        [h, parts],
        _mlp_layers(params["gine"][1]) + _mlp_layers(params["final_mlp"]),
        (True, True, False, True, True, False), block_rows=2000)


# speedup vs baseline: 2.1362x; 1.0699x over previous
import dataclasses
import functools

import jax
import jax.numpy as jnp
import numpy as np
from jax import lax
from jax.experimental import pallas as pl
from jax.experimental.pallas import tpu as pltpu
from jax.experimental.pallas import tpu_sc as plsc

_NC = 2
_NS = 16
_LN = 16
_W = 80
_SW = 25

_IDX = np.arange(128).reshape(4, 2, 16)
_PLO = _IDX[:, 0, :].reshape(-1)
_PHI = _IDX[:, 1, :].reshape(-1)


def _round_bf16_bits(v32):
    return v32 + 32767 + jnp.bitwise_and(jnp.right_shift(v32, 16), 1)


def _mlp_pallas(x_list, layers, relu_flags, block_rows, out_dtype=jnp.float32,
                mm_dtype=None, pack_out=None):
    rows = x_list[0].shape[0]
    for x in x_list:
        if x.ndim == 2:
            rows = x.shape[0]
    din = x_list[0].shape[-1]
    n_lay = len(layers)
    if pack_out is not None:
        dout = pack_out[0].shape[1]
        out_dtype = jnp.int32
    else:
        dout = layers[-1][0].shape[1]

    n_refs = sum(2 if x.ndim == 3 else 1 for x in x_list)
    wmats = [(w, b) for w, b in layers]
    if pack_out is not None:
        wmats += [(pack_out[0], pack_out[1]), (pack_out[2], pack_out[3])]

    def body(*refs):
        x_refs = refs[:n_refs]
        w_refs = refs[n_refs:n_refs + 2 * len(wmats)]
        acc = None
        for r in x_refs:
            v = r[0] if len(r.shape) == 3 else r[...]
            acc = v if acc is None else acc + v

        def mat(a, li):
            w = w_refs[2 * li][...]
            b = w_refs[2 * li + 1][...]
            if mm_dtype is not None:
                a = a.astype(mm_dtype)
                w = w.astype(mm_dtype)
            return jnp.dot(a, w, preferred_element_type=jnp.float32) + b

        for li in range(n_lay):
            acc = mat(acc, li)
            if relu_flags[li]:
                acc = jnp.maximum(acc, 0.0)
        if pack_out is not None:
            lo = jax.lax.bitcast_convert_type(mat(acc, n_lay), jnp.int32)
            hi = jax.lax.bitcast_convert_type(mat(acc, n_lay + 1), jnp.int32)
            lo_bits = jnp.bitwise_and(
                jnp.right_shift(_round_bf16_bits(lo), 16), 65535)
            hi_bits = jnp.bitwise_and(_round_bf16_bits(hi), -65536)
            refs[-1][...] = jnp.bitwise_or(lo_bits, hi_bits)
        else:
            refs[-1][...] = acc.astype(out_dtype)

    in_specs = []
    flat = []
    for x in x_list:
        if x.ndim == 3:
            for p in range(2):
                in_specs.append(pl.BlockSpec(
                    (1, block_rows, din), lambda i, p=p: (p, i, 0)))
                flat.append(x)
        else:
            in_specs.append(pl.BlockSpec((block_rows, din), lambda i: (i, 0)))
            flat.append(x)
    for w, b in wmats:
        in_specs.append(pl.BlockSpec(w.shape, lambda i: (0, 0)))
        in_specs.append(pl.BlockSpec((1, b.shape[0]), lambda i: (0, 0)))
        flat += [w, b.reshape(1, -1)]

    return pl.pallas_call(
        body,
        grid=(rows // block_rows,),
        in_specs=in_specs,
        out_specs=pl.BlockSpec((block_rows, dout), lambda i: (i, 0)),
        out_shape=jax.ShapeDtypeStruct((rows, dout), out_dtype),
    )(*flat)


def _mlp_layers(p):
    return [(p["W1"], p["b1"]), (p["W2"], p["b2"]), (p["W3"], p["b3"])]


def _pack_last(p):
    return (p["W3"][:, _PLO], p["b3"][_PLO],
            p["W3"][:, _PHI], p["b3"][_PHI])


def _sc_message(hp, ep, src5, dst5):
    n, d = hp.shape
    dw = ep.shape[1]
    nsw = src5.shape[2]
    epw = nsw * _SW * _W
    npad = -(-n // (_NS * 8)) * (_NS * 8)
    rps = npad // _NS
    nfull = rps // _W
    tail = rps - nfull * _W
    mesh = plsc.VectorSubcoreMesh(core_axis_name="c", subcore_axis_name="s")
    cp = pltpu.CompilerParams()
    if "needs_layout_passes" in pltpu.CompilerParams.__dataclass_fields__:
        cp = dataclasses.replace(cp, needs_layout_passes=False)

    @functools.partial(
        pl.kernel,
        out_type=jax.ShapeDtypeStruct((_NC, npad, d), jnp.float32),
        mesh=mesh,
        compiler_params=cp,
        scratch_types=[
            pltpu.VMEM((_SW, _W), jnp.int32),
            pltpu.VMEM((_SW, _W), jnp.int32),
            pltpu.VMEM((_W, d), jnp.float32),
            pltpu.VMEM((_W, dw), jnp.int32),
            pltpu.VMEM((_W, d), jnp.float32),
            pltpu.VMEM((_W, dw), jnp.int32),
            pltpu.VMEM_SHARED((npad, d), jnp.float32),
            pltpu.SemaphoreType.DMA,
            pltpu.SemaphoreType.DMA,
            pltpu.SemaphoreType.DMA,
            pltpu.SemaphoreType.DMA,
        ],
    )
    def k(h_hbm, e_hbm, src_hbm, dst_hbm, out_hbm,
          src_v, dst_v, h_a, e_a, h_b, e_b, agg,
          sem_ha, sem_ea, sem_hb, sem_eb):
        c = lax.axis_index("c")
        s = lax.axis_index("s")
        wbase = (c * _NS + s) * epw

        @pl.loop(0, _W)
        def _(i):
            for j in range(d // _LN):
                h_a[i, pl.ds(j * _LN, _LN)] = jnp.zeros((_LN,), jnp.float32)

        @pl.loop(0, nfull)
        def _(q):
            pltpu.async_copy(
                h_a, agg.at[pl.ds(s * rps + q * _W, _W), :], sem_ha)
        if tail:
            pltpu.async_copy(
                h_a.at[pl.ds(0, tail), :],
                agg.at[pl.ds(s * rps + nfull * _W, tail), :], sem_ea)

        @pl.loop(0, nfull)
        def _(q):
            pltpu.make_async_copy(
                h_a, agg.at[pl.ds(s * rps, _W), :], sem_ha).wait()
        if tail:
            pltpu.make_async_copy(
                h_a.at[pl.ds(0, tail), :],
                agg.at[pl.ds(s * rps, tail), :], sem_ea).wait()

        plsc.subcore_barrier()

        mask = jnp.full((16,), -65536, jnp.int32)

        def fire(sw, g, hbuf, ebuf, sem_h, sem_e):
            pltpu.async_copy(h_hbm.at[src_v.at[g]], hbuf, sem_h)
            pltpu.async_copy(
                e_hbm.at[pl.ds(wbase + (sw * _SW + g) * _W, _W), :],
                ebuf, sem_e)

        def drain(hbuf, ebuf, sem_h, sem_e):
            pltpu.make_async_copy(h_hbm.at[pl.ds(0, _W), :], hbuf, sem_h).wait()
            pltpu.make_async_copy(e_hbm.at[pl.ds(0, _W), :], ebuf, sem_e).wait()

        def compute_scatter(g, hbuf, ebuf):
            @plsc.parallel_loop(0, _W)
            def _(i):
                for j in range(d // 32):
                    w = ebuf[i, pl.ds(j * _LN, _LN)]
                    lo = plsc.bitcast(jnp.left_shift(w, 16), jnp.float32)
                    hi = plsc.bitcast(jnp.bitwise_and(w, mask), jnp.float32)
                    sl = pl.ds(j * 32, _LN)
                    sh = pl.ds(j * 32 + _LN, _LN)
                    hbuf[i, sl] = jnp.maximum(hbuf[i, sl] + lo, 0.0)
                    hbuf[i, sh] = jnp.maximum(hbuf[i, sh] + hi, 0.0)
            pltpu.sync_copy(hbuf, agg.at[dst_v.at[g]], add=True)

        @pl.loop(0, nsw)
        def _(sw):
            pltpu.sync_copy(src_hbm.at[c, s, sw], src_v)
            pltpu.sync_copy(dst_hbm.at[c, s, sw], dst_v)
            fire(sw, 0, h_a, e_a, sem_ha, sem_ea)
            fire(sw, 1, h_b, e_b, sem_hb, sem_eb)

            @pl.loop(0, _SW, step=2)
            def _(g):
                drain(h_a, e_a, sem_ha, sem_ea)
                compute_scatter(g, h_a, e_a)

                @pl.when(g + 2 < _SW)
                def _():
                    fire(sw, g + 2, h_a, e_a, sem_ha, sem_ea)

                @pl.when(g + 1 < _SW)
                def _():
                    drain(h_b, e_b, sem_hb, sem_eb)
                    compute_scatter(g + 1, h_b, e_b)

                    @pl.when(g + 3 < _SW)
                    def _():
                        fire(sw, g + 3, h_b, e_b, sem_hb, sem_eb)

        plsc.subcore_barrier()

        @pl.loop(0, nfull)
        def _(q):
            base = s * rps + q * _W
            pltpu.async_copy(agg.at[pl.ds(base, _W), :],
                             out_hbm.at[c, pl.ds(base, _W), :], sem_ha)
        if tail:
            base = s * rps + nfull * _W
            pltpu.async_copy(agg.at[pl.ds(base, tail), :],
                             out_hbm.at[c, pl.ds(base, tail), :], sem_ea)

        @pl.loop(0, nfull)
        def _(q):
            pltpu.make_async_copy(agg.at[pl.ds(s * rps, _W), :],
                                  out_hbm.at[c, pl.ds(s * rps, _W), :],
                                  sem_ha).wait()
        if tail:
            pltpu.make_async_copy(agg.at[pl.ds(s * rps, tail), :],
                                  out_hbm.at[c, pl.ds(s * rps, tail), :],
                                  sem_ea).wait()

    return k(hp, ep, src5, dst5)


def kernel(x, edge_index, edge_attr, params):
    e_cnt = edge_attr.shape[0]
    nsw = e_cnt // (_NC * _NS * _SW * _W)
    src5 = edge_index[0].astype(jnp.int32).reshape(_NC, _NS, nsw, _SW, _W)
    dst5 = edge_index[1].astype(jnp.int32).reshape(_NC, _NS, nsw, _SW, _W)

    h = _mlp_pallas([x], _mlp_layers(params["node_to_node"]),
                    (True, True, False), block_rows=2000)
    ep = _mlp_pallas([edge_attr], _mlp_layers(params["edge_to_node"])[:2],
                     (True, True), block_rows=2000,
                     mm_dtype=jnp.bfloat16,
                     pack_out=_pack_last(params["edge_to_node"]))

    parts = _sc_message(h, ep, src5, dst5)
    h = _mlp_pallas([h, parts], _mlp_layers(params["gine"][0]),
                    (True, True, False), block_rows=2000)

    parts = _sc_message(h, ep, src5, dst5)
    return _mlp_pallas(
        [h, parts],
        _mlp_layers(params["gine"][1]) + _mlp_layers(params["final_mlp"]),
        (True, True, False, True, True, False), block_rows=2000)
